# baseline (speedup 1.0000x reference)
import dataclasses
import functools

import jax
import jax.numpy as jnp
from jax import lax
from jax.experimental import pallas as pl
from jax.experimental.pallas import tpu as pltpu
from jax.experimental.pallas import tpu_sc as plsc

B = 4096
N = 100000
D = 64
K = 32
NCLS = 100
S = 2048
NCH = 49
NPAD = NCH * S
BQ = 256
INF = float("inf")


SUB = 256


def _tc_select_body(x_ref, st_ref, t_ref, w_ref, idx_ref, dm_ref, m_ref):
    xb = x_ref[...]
    x2 = jnp.sum(xb * xb, axis=1, keepdims=True)
    laneS = lax.broadcasted_iota(jnp.int32, (BQ, S), 1)
    laneSub = lax.broadcasted_iota(jnp.int32, (BQ, SUB), 1)
    laneK = lax.broadcasted_iota(jnp.int32, (BQ, K), 1)

    def chunk_d2(c):
        sb = st_ref[c]
        prod = jnp.dot(xb, sb, preferred_element_type=jnp.float32)
        s2 = jnp.sum(sb * sb, axis=0, keepdims=True)
        d2 = x2 + s2 - 2.0 * prod
        d2 = jnp.maximum(d2, 1e-12)
        valid = (c * S + laneS) < N
        return jnp.where(valid, d2, INF)

    m_ref[...] = jnp.full((BQ, 256), INF, jnp.float32)

    def pass_a(c, _):
        d2 = chunk_d2(c)
        m1 = m_ref[:, 0:128]
        m2 = m_ref[:, 128:256]
        for g in range(S // 128):
            x = d2[:, g * 128:(g + 1) * 128]
            m2 = jnp.minimum(m2, jnp.maximum(m1, x))
            m1 = jnp.minimum(m1, x)
        m_ref[:, 0:128] = m1
        m_ref[:, 128:256] = m2
        return 0

    lax.fori_loop(0, NCH, pass_a, 0)

    def kth(i, carry):
        mm, _ = carry
        mn = jnp.min(mm, axis=1, keepdims=True)
        mm = jnp.where(mm == mn, INF, mm)
        return mm, mn

    _, t0 = lax.fori_loop(0, K, kth, (m_ref[...], jnp.zeros((BQ, 1), jnp.float32)))

    def pass_b(c, carry):
        bd, bi = carry
        d2 = chunk_d2(c)
        dm_ref[...] = jnp.where(d2 <= t0, d2, INF)

        for q in range(S // SUB):
            lo = q * SUB
            dm0 = dm_ref[:, lo:lo + SUB]
            mn0 = jnp.min(dm0, axis=1, keepdims=True)
            cm0 = jnp.max(bd, axis=1, keepdims=True)
            go0 = jnp.any(mn0 < cm0)

            def cond(st):
                return st[4]

            def body(st):
                bd, bi, mn, _, _ = st
                dm = dm_ref[:, lo:lo + SUB]
                jl = jnp.min(jnp.where(dm == mn, laneSub, SUB), axis=1,
                             keepdims=True)
                cm = jnp.max(bd, axis=1, keepdims=True)
                acc = mn < cm
                pk = jnp.min(jnp.where(bd == cm, laneK, K), axis=1,
                             keepdims=True)
                hit = (laneK == pk) & acc
                bd = jnp.where(hit, mn, bd)
                bi = jnp.where(hit, c * S + lo + jl, bi)
                dm = jnp.where(laneSub == jl, INF, dm)
                dm_ref[:, lo:lo + SUB] = dm
                mn2 = jnp.min(dm, axis=1, keepdims=True)
                cm2 = jnp.max(bd, axis=1, keepdims=True)
                return bd, bi, mn2, cm2, jnp.any(mn2 < cm2)

            bd, bi, _, _, _ = lax.while_loop(
                cond, body, (bd, bi, mn0, cm0, go0))
        return bd, bi

    bd0 = jnp.full((BQ, K), INF, jnp.float32)
    bi0 = jnp.zeros((BQ, K), jnp.int32)
    bd, bi = lax.fori_loop(0, NCH, pass_b, (bd0, bi0))

    d = jnp.sqrt(bd)
    temp = t_ref[0, 0]
    logits = -d / temp
    mx = jnp.max(logits, axis=1, keepdims=True)
    e = jnp.exp(logits - mx)
    w = e / jnp.sum(e, axis=1, keepdims=True)
    w_ref[...] = w
    idx_ref[...] = bi


def _tc_select(x, st3, temp11):
    return pl.pallas_call(
        _tc_select_body,
        grid=(B // BQ,),
        in_specs=[
            pl.BlockSpec((BQ, D), lambda i: (i, 0)),
            pl.BlockSpec((NCH, D, S), lambda i: (0, 0, 0)),
            pl.BlockSpec(memory_space=pltpu.SMEM),
        ],
        out_specs=[
            pl.BlockSpec((BQ, K), lambda i: (i, 0)),
            pl.BlockSpec((BQ, K), lambda i: (i, 0)),
        ],
        out_shape=[
            jax.ShapeDtypeStruct((B, K), jnp.float32),
            jax.ShapeDtypeStruct((B, K), jnp.int32),
        ],
        scratch_shapes=[
            pltpu.VMEM((BQ, S), jnp.float32),
            pltpu.VMEM((BQ, 256), jnp.float32),
        ],
        compiler_params=pltpu.CompilerParams(
            vmem_limit_bytes=100 * 1024 * 1024,
        ),
    )(x, st3, temp11)


def _sc_combine(w_flat, idx_flat, labels):
    info = plsc.get_sparse_core_info()
    nc, ns = info.num_cores, info.num_subcores
    nw = nc * ns
    rw = B // nw
    rb = 32
    nblk = rw // rb
    mesh = plsc.VectorSubcoreMesh(core_axis_name="c", subcore_axis_name="s")
    cp = pltpu.CompilerParams()
    if "needs_layout_passes" in pltpu.CompilerParams.__dataclass_fields__:
        cp = dataclasses.replace(cp, needs_layout_passes=False)

    @functools.partial(
        pl.kernel,
        mesh=mesh,
        compiler_params=cp,
        out_type=jax.ShapeDtypeStruct((B * NCLS,), jnp.float32),
        scratch_types=[
            pltpu.VMEM((N,), jnp.int32),
            pltpu.VMEM((rb * K,), jnp.int32),
            pltpu.VMEM((rb * K,), jnp.float32),
            pltpu.VMEM((rb * NCLS,), jnp.float32),
        ],
    )
    def sc_kernel(w_hbm, idx_hbm, lab_hbm, out_hbm, labv, idxv, wv, pv):
        wid = lax.axis_index("s") * nc + lax.axis_index("c")
        pltpu.sync_copy(lab_hbm, labv)
        zero16 = jnp.zeros((16,), jnp.float32)

        def do_block(blk, _):
            base = (wid * rw + blk * rb)

            pltpu.sync_copy(idx_hbm.at[pl.ds(base * K, rb * K)], idxv)
            pltpu.sync_copy(w_hbm.at[pl.ds(base * K, rb * K)], wv)

            def zero_row(i, _):
                pv[pl.ds(i * 16, 16)] = zero16
                return 0

            lax.fori_loop(0, (rb * NCLS) // 16, zero_row, 0)

            def do_row(r, _):
                for g in range(K // 16):
                    iv = idxv[pl.ds(r * K + g * 16, 16)]
                    lab = plsc.load_gather(labv, [iv])
                    wvec = wv[pl.ds(r * K + g * 16, 16)]
                    flat = lab + r * NCLS
                    plsc.addupdate_scatter(pv, [flat], wvec)
                return 0

            lax.fori_loop(0, rb, do_row, 0)
            pltpu.sync_copy(pv, out_hbm.at[pl.ds(base * NCLS, rb * NCLS)])
            return 0

        lax.fori_loop(0, nblk, do_block, 0)

    return sc_kernel(w_flat, idx_flat, labels)


def kernel(x, support_embeddings, support_labels, temperature):
    sp = jnp.pad(support_embeddings, ((0, NPAD - N), (0, 0)))
    st3 = sp.reshape(NCH, S, D).transpose(0, 2, 1)
    temp11 = jnp.reshape(temperature, (1, 1)).astype(jnp.float32)
    w, idx = _tc_select(x, st3, temp11)
    probs_flat = _sc_combine(w.reshape(-1), idx.reshape(-1), support_labels)
    return probs_flat.reshape(B, NCLS)


# speedup vs baseline: 2.1267x; 2.1267x over previous
import dataclasses
import functools

import jax
import jax.numpy as jnp
from jax import lax
from jax.experimental import pallas as pl
from jax.experimental.pallas import tpu as pltpu
from jax.experimental.pallas import tpu_sc as plsc

B = 4096
N = 100000
D = 64
K = 32
NCLS = 100
S = 2048
NCH = 49
NPAD = NCH * S
BQ = 256
INF = float("inf")


def _tc_select_body(x_ref, st_ref, t_ref, w_ref, idx_ref, dm_ref, m_ref):
    xb = x_ref[...]
    x2 = jnp.sum(xb * xb, axis=1, keepdims=True)
    laneS = lax.broadcasted_iota(jnp.int32, (BQ, S), 1)
    lane1S = lax.broadcasted_iota(jnp.int32, (1, S), 1)
    laneK = lax.broadcasted_iota(jnp.int32, (BQ, K), 1)

    def chunk_e(c):
        sb = st_ref[c]
        prod = jnp.dot(xb, sb, preferred_element_type=jnp.float32)
        s2 = jnp.sum(sb * sb, axis=0, keepdims=True)
        s2m = jnp.where((c * S + lane1S) < N, s2, INF)
        return s2m - 2.0 * prod

    m_ref[...] = jnp.full((BQ, 256), INF, jnp.float32)

    def pass_a(c, _):
        e = chunk_e(c)
        m = m_ref[...]
        for g in range(S // 256):
            m = jnp.minimum(m, e[:, g * 256:(g + 1) * 256])
        m_ref[...] = m
        return 0

    lax.fori_loop(0, NCH, pass_a, 0)

    def kth(i, carry):
        mm, _ = carry
        mn = jnp.min(mm, axis=1, keepdims=True)
        mm = jnp.where(mm == mn, INF, mm)
        return mm, mn

    _, t0 = lax.fori_loop(0, K, kth, (m_ref[...], jnp.zeros((BQ, 1), jnp.float32)))

    def pass_b(c, carry):
        bd, bi = carry
        e = chunk_e(c)
        act = e <= t0
        dm_ref[...] = jnp.where(act, e, INF)
        cnt = jnp.sum(act.astype(jnp.int32), axis=1)
        mx = jnp.max(cnt)

        def step(i, st):
            bd, bi = st
            for _ in range(2):
                dm = dm_ref[...]
                mn = jnp.min(dm, axis=1, keepdims=True)
                jl = jnp.min(jnp.where(dm == mn, laneS, S), axis=1,
                             keepdims=True)
                cm = jnp.max(bd, axis=1, keepdims=True)
                acc = mn < cm
                pk = jnp.min(jnp.where(bd == cm, laneK, K), axis=1,
                             keepdims=True)
                hit = (laneK == pk) & acc
                bd = jnp.where(hit, mn, bd)
                bi = jnp.where(hit, c * S + jl, bi)
                dm_ref[...] = jnp.where(laneS == jl, INF, dm)
            return bd, bi

        return lax.fori_loop(0, (mx + 1) // 2, step, (bd, bi))

    bd0 = jnp.full((BQ, K), INF, jnp.float32)
    bi0 = jnp.zeros((BQ, K), jnp.int32)
    bd, bi = lax.fori_loop(0, NCH, pass_b, (bd0, bi0))

    d = jnp.sqrt(jnp.maximum(bd + x2, 1e-12))
    temp = t_ref[0, 0]
    logits = -d / temp
    mx = jnp.max(logits, axis=1, keepdims=True)
    e = jnp.exp(logits - mx)
    w = e / jnp.sum(e, axis=1, keepdims=True)
    w_ref[...] = w
    idx_ref[...] = bi


def _tc_select(x, st3, temp11):
    return pl.pallas_call(
        _tc_select_body,
        grid=(B // BQ,),
        in_specs=[
            pl.BlockSpec((BQ, D), lambda i: (i, 0)),
            pl.BlockSpec((NCH, D, S), lambda i: (0, 0, 0)),
            pl.BlockSpec(memory_space=pltpu.SMEM),
        ],
        out_specs=[
            pl.BlockSpec((BQ, K), lambda i: (i, 0)),
            pl.BlockSpec((BQ, K), lambda i: (i, 0)),
        ],
        out_shape=[
            jax.ShapeDtypeStruct((B, K), jnp.float32),
            jax.ShapeDtypeStruct((B, K), jnp.int32),
        ],
        scratch_shapes=[
            pltpu.VMEM((BQ, S), jnp.float32),
            pltpu.VMEM((BQ, 256), jnp.float32),
        ],
        compiler_params=pltpu.CompilerParams(
            vmem_limit_bytes=100 * 1024 * 1024,
        ),
    )(x, st3, temp11)


def _sc_combine(w_flat, idx_flat, labels):
    info = plsc.get_sparse_core_info()
    nc, ns = info.num_cores, info.num_subcores
    nw = nc * ns
    rw = B // nw
    rb = 32
    nblk = rw // rb
    mesh = plsc.VectorSubcoreMesh(core_axis_name="c", subcore_axis_name="s")
    cp = pltpu.CompilerParams()
    if "needs_layout_passes" in pltpu.CompilerParams.__dataclass_fields__:
        cp = dataclasses.replace(cp, needs_layout_passes=False)

    @functools.partial(
        pl.kernel,
        mesh=mesh,
        compiler_params=cp,
        out_type=jax.ShapeDtypeStruct((B * NCLS,), jnp.float32),
        scratch_types=[
            pltpu.VMEM((N,), jnp.int32),
            pltpu.VMEM((rb * K,), jnp.int32),
            pltpu.VMEM((rb * K,), jnp.float32),
            pltpu.VMEM((rb * NCLS,), jnp.float32),
        ],
    )
    def sc_kernel(w_hbm, idx_hbm, lab_hbm, out_hbm, labv, idxv, wv, pv):
        wid = lax.axis_index("s") * nc + lax.axis_index("c")
        pltpu.sync_copy(lab_hbm, labv)
        zero16 = jnp.zeros((16,), jnp.float32)

        def do_block(blk, _):
            base = (wid * rw + blk * rb)

            pltpu.sync_copy(idx_hbm.at[pl.ds(base * K, rb * K)], idxv)
            pltpu.sync_copy(w_hbm.at[pl.ds(base * K, rb * K)], wv)

            def zero_row(i, _):
                pv[pl.ds(i * 16, 16)] = zero16
                return 0

            lax.fori_loop(0, (rb * NCLS) // 16, zero_row, 0)

            def do_row(r, _):
                for g in range(K // 16):
                    iv = idxv[pl.ds(r * K + g * 16, 16)]
                    lab = plsc.load_gather(labv, [iv])
                    wvec = wv[pl.ds(r * K + g * 16, 16)]
                    flat = lab + r * NCLS
                    plsc.addupdate_scatter(pv, [flat], wvec)
                return 0

            lax.fori_loop(0, rb, do_row, 0)
            pltpu.sync_copy(pv, out_hbm.at[pl.ds(base * NCLS, rb * NCLS)])
            return 0

        lax.fori_loop(0, nblk, do_block, 0)

    return sc_kernel(w_flat, idx_flat, labels)


def kernel(x, support_embeddings, support_labels, temperature):
    sp = jnp.pad(support_embeddings, ((0, NPAD - N), (0, 0)))
    st3 = sp.reshape(NCH, S, D).transpose(0, 2, 1)
    temp11 = jnp.reshape(temperature, (1, 1)).astype(jnp.float32)
    w, idx = _tc_select(x, st3, temp11)
    probs_flat = _sc_combine(w.reshape(-1), idx.reshape(-1), support_labels)
    return probs_flat.reshape(B, NCLS)


# speedup vs baseline: 2.4492x; 1.1516x over previous
import dataclasses
import functools

import jax
import jax.numpy as jnp
from jax import lax
from jax.experimental import pallas as pl
from jax.experimental.pallas import tpu as pltpu
from jax.experimental.pallas import tpu_sc as plsc

B = 4096
N = 100000
D = 64
K = 32
NCLS = 100
S = 2048
NCH = 49
NPAD = NCH * S
BQ = 256
INF = float("inf")


def _tc_select_body(x_ref, st_ref, t_ref, w_ref, idx_ref, dm_ref, m_ref,
                    fi_ref):
    xb = x_ref[...]
    x2 = jnp.sum(xb * xb, axis=1, keepdims=True)
    laneS = lax.broadcasted_iota(jnp.int32, (BQ, S), 1)
    lane1S = lax.broadcasted_iota(jnp.int32, (1, S), 1)
    laneK = lax.broadcasted_iota(jnp.int32, (BQ, K), 1)

    def chunk_e(c):
        sb = st_ref[c]
        prod = jnp.dot(xb, sb, preferred_element_type=jnp.float32)
        s2 = jnp.sum(sb * sb, axis=0, keepdims=True)
        s2m = jnp.where((c * S + lane1S) < N, s2, INF)
        return s2m - 2.0 * prod

    m_ref[...] = jnp.full((BQ, 256), INF, jnp.float32)

    def pass_a(c, _):
        e = chunk_e(c)
        m = m_ref[...]
        for g in range(S // 256):
            m = jnp.minimum(m, e[:, g * 256:(g + 1) * 256])
        m_ref[...] = m
        return 0

    lax.fori_loop(0, NCH, pass_a, 0)

    def kth(i, carry):
        mm, _ = carry
        mn = jnp.min(mm, axis=1, keepdims=True)
        mm = jnp.where(mm == mn, INF, mm)
        return mm, mn

    _, t0 = lax.fori_loop(0, K, kth, (m_ref[...], jnp.zeros((BQ, 1), jnp.float32)))

    NG = S // 256
    lane256 = lax.broadcasted_iota(jnp.int32, (BQ, 256), 1)

    def pass_b(c, carry):
        bd, bi = carry
        e = chunk_e(c)
        act = e <= t0
        dm = jnp.where(act, e, INF)
        dm_ref[...] = dm

        fv = jnp.full((BQ, 256), INF, jnp.float32)
        fi = jnp.zeros((BQ, 256), jnp.int32)
        cc = jnp.zeros((BQ, 256), jnp.int32)
        for g in range(NG):
            xg = dm[:, g * 256:(g + 1) * 256]
            upd = xg < fv
            fv = jnp.where(upd, xg, fv)
            fi = jnp.where(upd, g, fi)
            cc = cc + act[:, g * 256:(g + 1) * 256].astype(jnp.int32)
        m_ref[...] = fv
        fi_ref[...] = fi
        dcnt = jnp.sum((cc > 0).astype(jnp.int32), axis=1, keepdims=True)
        cnt = jnp.sum(cc, axis=1, keepdims=True)
        mx2 = jnp.max(dcnt)
        mxm = jnp.max(cnt - dcnt)

        def step2(i, st):
            bd, bi = st
            fv = m_ref[...]
            mn = jnp.min(fv, axis=1, keepdims=True)
            jl = jnp.min(jnp.where(fv == mn, lane256, 256), axis=1,
                         keepdims=True)
            gsel = jnp.sum(jnp.where(lane256 == jl, fi_ref[...], 0), axis=1,
                           keepdims=True)
            cm = jnp.max(bd, axis=1, keepdims=True)
            acc = mn < cm
            pk = jnp.min(jnp.where(bd == cm, laneK, K), axis=1, keepdims=True)
            hit = (laneK == pk) & acc
            bd = jnp.where(hit, mn, bd)
            bi = jnp.where(hit, c * S + gsel * 256 + jl, bi)
            m_ref[...] = jnp.where(lane256 == jl, INF, fv)
            return bd, bi

        bd, bi = lax.fori_loop(0, mx2, step2, (bd, bi))

        def step3(i, st):
            bd, bi = st
            dm = dm_ref[...]
            mn = jnp.min(dm, axis=1, keepdims=True)
            jl = jnp.min(jnp.where(dm == mn, laneS, S), axis=1, keepdims=True)
            cm = jnp.max(bd, axis=1, keepdims=True)
            acc = mn < cm
            pk = jnp.min(jnp.where(bd == cm, laneK, K), axis=1, keepdims=True)
            hit = (laneK == pk) & acc
            bd = jnp.where(hit, mn, bd)
            bi = jnp.where(hit, c * S + jl, bi)
            dm_ref[...] = jnp.where(laneS == jl, INF, dm)
            return bd, bi

        fi0 = fi_ref[...]
        fi_t = jnp.concatenate([fi0] * NG, axis=1)
        giota = laneS >> 8
        dm_ref[...] = jnp.where(giota != fi_t, dm_ref[...], INF)
        return lax.fori_loop(0, mxm, step3, (bd, bi))

    bd0 = jnp.full((BQ, K), INF, jnp.float32)
    bi0 = jnp.zeros((BQ, K), jnp.int32)
    bd, bi = lax.fori_loop(0, NCH, pass_b, (bd0, bi0))

    d = jnp.sqrt(jnp.maximum(bd + x2, 1e-12))
    temp = t_ref[0, 0]
    logits = -d / temp
    mx = jnp.max(logits, axis=1, keepdims=True)
    e = jnp.exp(logits - mx)
    w = e / jnp.sum(e, axis=1, keepdims=True)
    w_ref[...] = w
    idx_ref[...] = bi


def _tc_select(x, st3, temp11):
    return pl.pallas_call(
        _tc_select_body,
        grid=(B // BQ,),
        in_specs=[
            pl.BlockSpec((BQ, D), lambda i: (i, 0)),
            pl.BlockSpec((NCH, D, S), lambda i: (0, 0, 0)),
            pl.BlockSpec(memory_space=pltpu.SMEM),
        ],
        out_specs=[
            pl.BlockSpec((BQ, K), lambda i: (i, 0)),
            pl.BlockSpec((BQ, K), lambda i: (i, 0)),
        ],
        out_shape=[
            jax.ShapeDtypeStruct((B, K), jnp.float32),
            jax.ShapeDtypeStruct((B, K), jnp.int32),
        ],
        scratch_shapes=[
            pltpu.VMEM((BQ, S), jnp.float32),
            pltpu.VMEM((BQ, 256), jnp.float32),
            pltpu.VMEM((BQ, 256), jnp.int32),
        ],
        compiler_params=pltpu.CompilerParams(
            vmem_limit_bytes=100 * 1024 * 1024,
        ),
    )(x, st3, temp11)


def _sc_combine(w_flat, idx_flat, labels):
    info = plsc.get_sparse_core_info()
    nc, ns = info.num_cores, info.num_subcores
    nw = nc * ns
    rw = B // nw
    rb = 32
    nblk = rw // rb
    mesh = plsc.VectorSubcoreMesh(core_axis_name="c", subcore_axis_name="s")
    cp = pltpu.CompilerParams()
    if "needs_layout_passes" in pltpu.CompilerParams.__dataclass_fields__:
        cp = dataclasses.replace(cp, needs_layout_passes=False)

    @functools.partial(
        pl.kernel,
        mesh=mesh,
        compiler_params=cp,
        out_type=jax.ShapeDtypeStruct((B * NCLS,), jnp.float32),
        scratch_types=[
            pltpu.VMEM((N,), jnp.int32),
            pltpu.VMEM((rb * K,), jnp.int32),
            pltpu.VMEM((rb * K,), jnp.float32),
            pltpu.VMEM((rb * NCLS,), jnp.float32),
        ],
    )
    def sc_kernel(w_hbm, idx_hbm, lab_hbm, out_hbm, labv, idxv, wv, pv):
        wid = lax.axis_index("s") * nc + lax.axis_index("c")
        pltpu.sync_copy(lab_hbm, labv)
        zero16 = jnp.zeros((16,), jnp.float32)

        def do_block(blk, _):
            base = (wid * rw + blk * rb)

            pltpu.sync_copy(idx_hbm.at[pl.ds(base * K, rb * K)], idxv)
            pltpu.sync_copy(w_hbm.at[pl.ds(base * K, rb * K)], wv)

            def zero_row(i, _):
                pv[pl.ds(i * 16, 16)] = zero16
                return 0

            lax.fori_loop(0, (rb * NCLS) // 16, zero_row, 0)

            def do_row(r, _):
                for g in range(K // 16):
                    iv = idxv[pl.ds(r * K + g * 16, 16)]
                    lab = plsc.load_gather(labv, [iv])
                    wvec = wv[pl.ds(r * K + g * 16, 16)]
                    flat = lab + r * NCLS
                    plsc.addupdate_scatter(pv, [flat], wvec)
                return 0

            lax.fori_loop(0, rb, do_row, 0)
            pltpu.sync_copy(pv, out_hbm.at[pl.ds(base * NCLS, rb * NCLS)])
            return 0

        lax.fori_loop(0, nblk, do_block, 0)

    return sc_kernel(w_flat, idx_flat, labels)


def kernel(x, support_embeddings, support_labels, temperature):
    sp = jnp.pad(support_embeddings, ((0, NPAD - N), (0, 0)))
    st3 = sp.reshape(NCH, S, D).transpose(0, 2, 1)
    temp11 = jnp.reshape(temperature, (1, 1)).astype(jnp.float32)
    w, idx = _tc_select(x, st3, temp11)
    probs_flat = _sc_combine(w.reshape(-1), idx.reshape(-1), support_labels)
    return probs_flat.reshape(B, NCLS)


# speedup vs baseline: 2.4969x; 1.0195x over previous
import dataclasses
import functools

import jax
import jax.numpy as jnp
from jax import lax
from jax.experimental import pallas as pl
from jax.experimental.pallas import tpu as pltpu
from jax.experimental.pallas import tpu_sc as plsc

B = 4096
N = 100000
D = 64
K = 32
NCLS = 100
S = 2048
NCH = 49
NPAD = NCH * S
BQ = 256
INF = float("inf")


def _tc_select_body(x_ref, st_ref, t_ref, w_ref, idx_ref, dm_ref, m_ref,
                    fi_ref, s2_ref):
    xb = x_ref[...]
    x2 = jnp.sum(xb * xb, axis=1, keepdims=True)
    laneS = lax.broadcasted_iota(jnp.int32, (BQ, S), 1)
    lane1S = lax.broadcasted_iota(jnp.int32, (1, S), 1)
    laneK = lax.broadcasted_iota(jnp.int32, (BQ, K), 1)

    def prep_s2(c, _):
        sb = st_ref[c]
        s2 = jnp.sum(sb * sb, axis=0, keepdims=True)
        s2_ref[c] = jnp.where((c * S + lane1S) < N, s2, INF)
        return 0

    lax.fori_loop(0, NCH, prep_s2, 0)

    def chunk_e(c):
        sb = st_ref[c]
        prod = jnp.dot(xb, sb, preferred_element_type=jnp.float32)
        return s2_ref[c] - 2.0 * prod

    m_ref[...] = jnp.full((BQ, 256), INF, jnp.float32)

    def pass_a(c, _):
        e = chunk_e(c)
        m = m_ref[...]
        for g in range(S // 256):
            m = jnp.minimum(m, e[:, g * 256:(g + 1) * 256])
        m_ref[...] = m
        return 0

    lax.fori_loop(0, NCH, pass_a, 0)

    def kth(i, carry):
        mm, _ = carry
        mn = jnp.min(mm, axis=1, keepdims=True)
        mm = jnp.where(mm == mn, INF, mm)
        return mm, mn

    _, t0 = lax.fori_loop(0, K, kth, (m_ref[...], jnp.zeros((BQ, 1), jnp.float32)))

    NG = S // 256
    lane256 = lax.broadcasted_iota(jnp.int32, (BQ, 256), 1)

    def pass_b(c, carry):
        bd, bi = carry
        e = chunk_e(c)
        act = e <= t0
        dm = jnp.where(act, e, INF)
        dm_ref[...] = dm

        fv = jnp.full((BQ, 256), INF, jnp.float32)
        fi = jnp.zeros((BQ, 256), jnp.int32)
        for g in range(NG):
            xg = dm[:, g * 256:(g + 1) * 256]
            upd = xg < fv
            fv = jnp.where(upd, xg, fv)
            fi = jnp.where(upd, g, fi)
        m_ref[...] = fv
        fi_ref[...] = fi
        cnt = jnp.sum(act.astype(jnp.int32), axis=1, keepdims=True)
        dcnt = jnp.sum((fv < INF).astype(jnp.int32), axis=1, keepdims=True)
        mx2 = jnp.max(dcnt)
        mxm = jnp.max(cnt - dcnt)

        def step2(i, st):
            bd, bi = st
            fv = m_ref[...]
            mn = jnp.min(fv, axis=1, keepdims=True)
            jl = jnp.min(jnp.where(fv == mn, lane256, 256), axis=1,
                         keepdims=True)
            gsel = jnp.sum(jnp.where(lane256 == jl, fi_ref[...], 0), axis=1,
                           keepdims=True)
            cm = jnp.max(bd, axis=1, keepdims=True)
            acc = mn < cm
            pk = jnp.min(jnp.where(bd == cm, laneK, K), axis=1, keepdims=True)
            hit = (laneK == pk) & acc
            bd = jnp.where(hit, mn, bd)
            bi = jnp.where(hit, c * S + gsel * 256 + jl, bi)
            m_ref[...] = jnp.where(lane256 == jl, INF, fv)
            return bd, bi

        bd, bi = lax.fori_loop(0, mx2, step2, (bd, bi))

        def step3(i, st):
            bd, bi = st
            dm = dm_ref[...]
            mn = jnp.min(dm, axis=1, keepdims=True)
            jl = jnp.min(jnp.where(dm == mn, laneS, S), axis=1, keepdims=True)
            cm = jnp.max(bd, axis=1, keepdims=True)
            acc = mn < cm
            pk = jnp.min(jnp.where(bd == cm, laneK, K), axis=1, keepdims=True)
            hit = (laneK == pk) & acc
            bd = jnp.where(hit, mn, bd)
            bi = jnp.where(hit, c * S + jl, bi)
            dm_ref[...] = jnp.where(laneS == jl, INF, dm)
            return bd, bi

        def cleanup(args):
            bd, bi = args
            fi_t = jnp.concatenate([fi_ref[...]] * NG, axis=1)
            giota = laneS >> 8
            dm_ref[...] = jnp.where(giota != fi_t, dm_ref[...], INF)
            return lax.fori_loop(0, mxm, step3, (bd, bi))

        return lax.cond(mxm > 0, cleanup, lambda args: args, (bd, bi))

    bd0 = jnp.full((BQ, K), INF, jnp.float32)
    bi0 = jnp.zeros((BQ, K), jnp.int32)
    bd, bi = lax.fori_loop(0, NCH, pass_b, (bd0, bi0))

    d = jnp.sqrt(jnp.maximum(bd + x2, 1e-12))
    temp = t_ref[0, 0]
    logits = -d / temp
    mx = jnp.max(logits, axis=1, keepdims=True)
    e = jnp.exp(logits - mx)
    w = e / jnp.sum(e, axis=1, keepdims=True)
    w_ref[...] = w
    idx_ref[...] = bi


def _tc_select(x, st3, temp11):
    return pl.pallas_call(
        _tc_select_body,
        grid=(B // BQ,),
        in_specs=[
            pl.BlockSpec((BQ, D), lambda i: (i, 0)),
            pl.BlockSpec((NCH, D, S), lambda i: (0, 0, 0)),
            pl.BlockSpec(memory_space=pltpu.SMEM),
        ],
        out_specs=[
            pl.BlockSpec((BQ, K), lambda i: (i, 0)),
            pl.BlockSpec((BQ, K), lambda i: (i, 0)),
        ],
        out_shape=[
            jax.ShapeDtypeStruct((B, K), jnp.float32),
            jax.ShapeDtypeStruct((B, K), jnp.int32),
        ],
        scratch_shapes=[
            pltpu.VMEM((BQ, S), jnp.float32),
            pltpu.VMEM((BQ, 256), jnp.float32),
            pltpu.VMEM((BQ, 256), jnp.int32),
            pltpu.VMEM((NCH, 1, S), jnp.float32),
        ],
        compiler_params=pltpu.CompilerParams(
            vmem_limit_bytes=100 * 1024 * 1024,
        ),
    )(x, st3, temp11)


def _sc_combine(w_flat, idx_flat, labels):
    info = plsc.get_sparse_core_info()
    nc, ns = info.num_cores, info.num_subcores
    nw = nc * ns
    rw = B // nw
    rb = 32
    nblk = rw // rb
    mesh = plsc.VectorSubcoreMesh(core_axis_name="c", subcore_axis_name="s")
    cp = pltpu.CompilerParams()
    if "needs_layout_passes" in pltpu.CompilerParams.__dataclass_fields__:
        cp = dataclasses.replace(cp, needs_layout_passes=False)

    @functools.partial(
        pl.kernel,
        mesh=mesh,
        compiler_params=cp,
        out_type=jax.ShapeDtypeStruct((B * NCLS,), jnp.float32),
        scratch_types=[
            pltpu.VMEM((N,), jnp.int32),
            pltpu.VMEM((rb * K,), jnp.int32),
            pltpu.VMEM((rb * K,), jnp.float32),
            pltpu.VMEM((rb * NCLS,), jnp.float32),
        ],
    )
    def sc_kernel(w_hbm, idx_hbm, lab_hbm, out_hbm, labv, idxv, wv, pv):
        wid = lax.axis_index("s") * nc + lax.axis_index("c")
        pltpu.sync_copy(lab_hbm, labv)
        zero16 = jnp.zeros((16,), jnp.float32)

        def do_block(blk, _):
            base = (wid * rw + blk * rb)

            pltpu.sync_copy(idx_hbm.at[pl.ds(base * K, rb * K)], idxv)
            pltpu.sync_copy(w_hbm.at[pl.ds(base * K, rb * K)], wv)

            def zero_row(i, _):
                pv[pl.ds(i * 16, 16)] = zero16
                return 0

            lax.fori_loop(0, (rb * NCLS) // 16, zero_row, 0)

            def do_row(r, _):
                for g in range(K // 16):
                    iv = idxv[pl.ds(r * K + g * 16, 16)]
                    lab = plsc.load_gather(labv, [iv])
                    wvec = wv[pl.ds(r * K + g * 16, 16)]
                    flat = lab + r * NCLS
                    plsc.addupdate_scatter(pv, [flat], wvec)
                return 0

            lax.fori_loop(0, rb, do_row, 0)
            pltpu.sync_copy(pv, out_hbm.at[pl.ds(base * NCLS, rb * NCLS)])
            return 0

        lax.fori_loop(0, nblk, do_block, 0)

    return sc_kernel(w_flat, idx_flat, labels)


def kernel(x, support_embeddings, support_labels, temperature):
    sp = jnp.pad(support_embeddings, ((0, NPAD - N), (0, 0)))
    st3 = sp.reshape(NCH, S, D).transpose(0, 2, 1)
    temp11 = jnp.reshape(temperature, (1, 1)).astype(jnp.float32)
    w, idx = _tc_select(x, st3, temp11)
    probs_flat = _sc_combine(w.reshape(-1), idx.reshape(-1), support_labels)
    return probs_flat.reshape(B, NCLS)


# speedup vs baseline: 2.6169x; 1.0480x over previous
import dataclasses
import functools

import jax
import jax.numpy as jnp
from jax import lax
from jax.experimental import pallas as pl
from jax.experimental.pallas import tpu as pltpu
from jax.experimental.pallas import tpu_sc as plsc

B = 4096
N = 100000
D = 64
K = 32
NCLS = 100
S = 2048
NCH = 50
NPAD = NCH * S
BQ = 256
INF = float("inf")


def _tc_select_body(x_ref, st_ref, t_ref, w_ref, idx_ref, dm_ref, m_ref,
                    fi_ref, s2_ref):
    xb = x_ref[...]
    x2 = jnp.sum(xb * xb, axis=1, keepdims=True)
    laneS = lax.broadcasted_iota(jnp.int32, (BQ, S), 1)
    lane1S = lax.broadcasted_iota(jnp.int32, (1, S), 1)
    laneK = lax.broadcasted_iota(jnp.int32, (BQ, K), 1)

    def prep_s2(c, _):
        sb = st_ref[c]
        s2 = jnp.sum(sb * sb, axis=0, keepdims=True)
        s2_ref[c] = jnp.where((c * S + lane1S) < N, s2, INF)
        return 0

    lax.fori_loop(0, NCH, prep_s2, 0)

    def chunk_e(c):
        sb = st_ref[c]
        prod = jnp.dot(xb, sb, preferred_element_type=jnp.float32)
        return s2_ref[c] - 2.0 * prod

    m_ref[...] = jnp.full((BQ, 512), INF, jnp.float32)

    def pass_a(c, _):
        e = chunk_e(c)
        m = m_ref[...]
        for g in range(S // 512):
            m = jnp.minimum(m, e[:, g * 512:(g + 1) * 512])
        m_ref[...] = m
        return 0

    lax.fori_loop(0, NCH, pass_a, 0)

    def kth(i, carry):
        mm, _ = carry
        mn = jnp.min(mm, axis=1, keepdims=True)
        mm = jnp.where(mm == mn, INF, mm)
        return mm, mn

    _, t0 = lax.fori_loop(0, K, kth, (m_ref[...], jnp.zeros((BQ, 1), jnp.float32)))

    NG = S // 256
    NP = NCH // 2
    S2 = 2 * S
    lane512 = lax.broadcasted_iota(jnp.int32, (BQ, 512), 1)
    laneS2 = lax.broadcasted_iota(jnp.int32, (BQ, S2), 1)

    def pass_b(p, carry):
        bd, bi = carry
        fvs, fis = [], []
        cnt = jnp.zeros((BQ, 1), jnp.int32)
        for h in range(2):
            e = chunk_e(2 * p + h)
            act = e <= t0
            dm = jnp.where(act, e, INF)
            dm_ref[:, h * S:(h + 1) * S] = dm
            fvh = jnp.full((BQ, 256), INF, jnp.float32)
            fih = jnp.zeros((BQ, 256), jnp.int32)
            for g in range(NG):
                xg = dm[:, g * 256:(g + 1) * 256]
                upd = xg < fvh
                fvh = jnp.where(upd, xg, fvh)
                fih = jnp.where(upd, g, fih)
            fvs.append(fvh)
            fis.append(fih)
            cnt = cnt + jnp.sum(act.astype(jnp.int32), axis=1, keepdims=True)
        fv = jnp.concatenate(fvs, axis=1)
        fi = jnp.concatenate(fis, axis=1)
        m_ref[...] = fv
        fi_ref[...] = fi
        dcnt = jnp.sum((fv < INF).astype(jnp.int32), axis=1, keepdims=True)
        mx2 = jnp.max(dcnt)
        mxm = jnp.max(cnt - dcnt)

        def step2(i, st):
            bd, bi = st
            for _ in range(2):
                fv = m_ref[...]
                mn = jnp.min(fv, axis=1, keepdims=True)
                jl = jnp.min(jnp.where(fv == mn, lane512, 512), axis=1,
                             keepdims=True)
                gsel = jnp.sum(jnp.where(lane512 == jl, fi_ref[...], 0),
                               axis=1, keepdims=True)
                cm = jnp.max(bd, axis=1, keepdims=True)
                acc = mn < cm
                pk = jnp.min(jnp.where(bd == cm, laneK, K), axis=1,
                             keepdims=True)
                hit = (laneK == pk) & acc
                half = (jl >= 256).astype(jnp.int32)
                gidx = 2 * p * S + half * S + gsel * 256 + (jl & 255)
                bd = jnp.where(hit, mn, bd)
                bi = jnp.where(hit, gidx, bi)
                m_ref[...] = jnp.where(lane512 == jl, INF, fv)
            return bd, bi

        bd, bi = lax.fori_loop(0, (mx2 + 1) // 2, step2, (bd, bi))

        def step3(i, st):
            bd, bi = st
            for _ in range(2):
                dm = dm_ref[...]
                mn = jnp.min(dm, axis=1, keepdims=True)
                jl = jnp.min(jnp.where(dm == mn, laneS2, S2), axis=1,
                             keepdims=True)
                cm = jnp.max(bd, axis=1, keepdims=True)
                acc = mn < cm
                pk = jnp.min(jnp.where(bd == cm, laneK, K), axis=1,
                             keepdims=True)
                hit = (laneK == pk) & acc
                bd = jnp.where(hit, mn, bd)
                bi = jnp.where(hit, 2 * p * S + jl, bi)
                dm_ref[...] = jnp.where(laneS2 == jl, INF, dm)
            return bd, bi

        def cleanup(args):
            bd, bi = args
            fi0 = fi_ref[...]
            parts = [fi0[:, 0:256]] * NG + [fi0[:, 256:512]] * NG
            fi_t = jnp.concatenate(parts, axis=1)
            giota = (laneS2 & (S - 1)) >> 8
            dm_ref[...] = jnp.where(giota != fi_t, dm_ref[...], INF)
            return lax.fori_loop(0, (mxm + 1) // 2, step3, (bd, bi))

        return lax.cond(mxm > 0, cleanup, lambda args: args, (bd, bi))

    bd0 = jnp.full((BQ, K), INF, jnp.float32)
    bi0 = jnp.zeros((BQ, K), jnp.int32)
    bd, bi = lax.fori_loop(0, NP, pass_b, (bd0, bi0))

    d = jnp.sqrt(jnp.maximum(bd + x2, 1e-12))
    temp = t_ref[0, 0]
    logits = -d / temp
    mx = jnp.max(logits, axis=1, keepdims=True)
    e = jnp.exp(logits - mx)
    w = e / jnp.sum(e, axis=1, keepdims=True)
    w_ref[...] = w
    idx_ref[...] = bi


def _tc_select(x, st3, temp11):
    return pl.pallas_call(
        _tc_select_body,
        grid=(B // BQ,),
        in_specs=[
            pl.BlockSpec((BQ, D), lambda i: (i, 0)),
            pl.BlockSpec((NCH, D, S), lambda i: (0, 0, 0)),
            pl.BlockSpec(memory_space=pltpu.SMEM),
        ],
        out_specs=[
            pl.BlockSpec((BQ, K), lambda i: (i, 0)),
            pl.BlockSpec((BQ, K), lambda i: (i, 0)),
        ],
        out_shape=[
            jax.ShapeDtypeStruct((B, K), jnp.float32),
            jax.ShapeDtypeStruct((B, K), jnp.int32),
        ],
        scratch_shapes=[
            pltpu.VMEM((BQ, 2 * S), jnp.float32),
            pltpu.VMEM((BQ, 512), jnp.float32),
            pltpu.VMEM((BQ, 512), jnp.int32),
            pltpu.VMEM((NCH, 1, S), jnp.float32),
        ],
        compiler_params=pltpu.CompilerParams(
            vmem_limit_bytes=100 * 1024 * 1024,
        ),
    )(x, st3, temp11)


def _sc_combine(w_flat, idx_flat, labels):
    info = plsc.get_sparse_core_info()
    nc, ns = info.num_cores, info.num_subcores
    nw = nc * ns
    rw = B // nw
    rb = 32
    nblk = rw // rb
    mesh = plsc.VectorSubcoreMesh(core_axis_name="c", subcore_axis_name="s")
    cp = pltpu.CompilerParams()
    if "needs_layout_passes" in pltpu.CompilerParams.__dataclass_fields__:
        cp = dataclasses.replace(cp, needs_layout_passes=False)

    @functools.partial(
        pl.kernel,
        mesh=mesh,
        compiler_params=cp,
        out_type=jax.ShapeDtypeStruct((B * NCLS,), jnp.float32),
        scratch_types=[
            pltpu.VMEM((N,), jnp.int32),
            pltpu.VMEM((rb * K,), jnp.int32),
            pltpu.VMEM((rb * K,), jnp.float32),
            pltpu.VMEM((rb * NCLS,), jnp.float32),
        ],
    )
    def sc_kernel(w_hbm, idx_hbm, lab_hbm, out_hbm, labv, idxv, wv, pv):
        wid = lax.axis_index("s") * nc + lax.axis_index("c")
        pltpu.sync_copy(lab_hbm, labv)
        zero16 = jnp.zeros((16,), jnp.float32)

        def do_block(blk, _):
            base = (wid * rw + blk * rb)

            pltpu.sync_copy(idx_hbm.at[pl.ds(base * K, rb * K)], idxv)
            pltpu.sync_copy(w_hbm.at[pl.ds(base * K, rb * K)], wv)

            def zero_row(i, _):
                pv[pl.ds(i * 16, 16)] = zero16
                return 0

            lax.fori_loop(0, (rb * NCLS) // 16, zero_row, 0)

            def do_row(r, _):
                for g in range(K // 16):
                    iv = idxv[pl.ds(r * K + g * 16, 16)]
                    lab = plsc.load_gather(labv, [iv])
                    wvec = wv[pl.ds(r * K + g * 16, 16)]
                    flat = lab + r * NCLS
                    plsc.addupdate_scatter(pv, [flat], wvec)
                return 0

            lax.fori_loop(0, rb, do_row, 0)
            pltpu.sync_copy(pv, out_hbm.at[pl.ds(base * NCLS, rb * NCLS)])
            return 0

        lax.fori_loop(0, nblk, do_block, 0)

    return sc_kernel(w_flat, idx_flat, labels)


def kernel(x, support_embeddings, support_labels, temperature):
    sp = jnp.pad(support_embeddings, ((0, NPAD - N), (0, 0)))
    st3 = sp.reshape(NCH, S, D).transpose(0, 2, 1)
    temp11 = jnp.reshape(temperature, (1, 1)).astype(jnp.float32)
    w, idx = _tc_select(x, st3, temp11)
    probs_flat = _sc_combine(w.reshape(-1), idx.reshape(-1), support_labels)
    return probs_flat.reshape(B, NCLS)


# speedup vs baseline: 2.6342x; 1.0066x over previous
import dataclasses
import functools

import jax
import jax.numpy as jnp
from jax import lax
from jax.experimental import pallas as pl
from jax.experimental.pallas import tpu as pltpu
from jax.experimental.pallas import tpu_sc as plsc

B = 4096
N = 100000
D = 64
K = 32
NCLS = 100
S = 2048
NCH = 50
NPAD = NCH * S
BQ = 256
INF = float("inf")


def _tc_select_body(x_ref, st_ref, t_ref, w_ref, idx_ref, dm_ref, m_ref,
                    fi_ref, s2_ref):
    xb = x_ref[...]
    x2 = jnp.sum(xb * xb, axis=1, keepdims=True)
    laneS = lax.broadcasted_iota(jnp.int32, (BQ, S), 1)
    lane1S = lax.broadcasted_iota(jnp.int32, (1, S), 1)
    laneK = lax.broadcasted_iota(jnp.int32, (BQ, K), 1)

    def prep_s2(c, _):
        sb = st_ref[c]
        s2 = jnp.sum(sb * sb, axis=0, keepdims=True)
        s2_ref[c] = jnp.where((c * S + lane1S) < N, s2, INF)
        return 0

    lax.fori_loop(0, NCH, prep_s2, 0)

    def chunk_e(c):
        sb = st_ref[c]
        prod = jnp.dot(xb, sb, preferred_element_type=jnp.float32)
        return s2_ref[c] - 2.0 * prod

    m_ref[...] = jnp.full((BQ, 512), INF, jnp.float32)

    def pass_a(c, _):
        e = chunk_e(c)
        m = m_ref[...]
        for g in range(S // 512):
            m = jnp.minimum(m, e[:, g * 512:(g + 1) * 512])
        m_ref[...] = m
        return 0

    lax.fori_loop(0, NCH, pass_a, 0)

    def kth(i, carry):
        mm, _ = carry
        mn = jnp.min(mm, axis=1, keepdims=True)
        mm = jnp.where(mm == mn, INF, mm)
        return mm, mn

    _, t0 = lax.fori_loop(0, K, kth, (m_ref[...], jnp.zeros((BQ, 1), jnp.float32)))

    NG = S // 256
    NP = NCH // 2
    S2 = 2 * S
    lane512 = lax.broadcasted_iota(jnp.int32, (BQ, 512), 1)
    laneS2 = lax.broadcasted_iota(jnp.int32, (BQ, S2), 1)

    def pass_b(p, carry):
        bd, bi = carry
        thr = jnp.minimum(t0, jnp.max(bd, axis=1, keepdims=True))
        fvs, fis = [], []
        cnt = jnp.zeros((BQ, 1), jnp.int32)
        for h in range(2):
            e = chunk_e(2 * p + h)
            act = e <= thr
            dm = jnp.where(act, e, INF)
            dm_ref[:, h * S:(h + 1) * S] = dm
            fvh = jnp.full((BQ, 256), INF, jnp.float32)
            fih = jnp.zeros((BQ, 256), jnp.int32)
            for g in range(NG):
                xg = dm[:, g * 256:(g + 1) * 256]
                upd = xg < fvh
                fvh = jnp.where(upd, xg, fvh)
                fih = jnp.where(upd, g, fih)
            fvs.append(fvh)
            fis.append(fih)
            cnt = cnt + jnp.sum(act.astype(jnp.int32), axis=1, keepdims=True)
        fv = jnp.concatenate(fvs, axis=1)
        fi = jnp.concatenate(fis, axis=1)
        m_ref[...] = fv
        fi_ref[...] = fi
        dcnt = jnp.sum((fv < INF).astype(jnp.int32), axis=1, keepdims=True)
        mx2 = jnp.max(dcnt)
        mxm = jnp.max(cnt - dcnt)

        def step2(i, st):
            bd, bi = st
            for _ in range(2):
                fv = m_ref[...]
                mn = jnp.min(fv, axis=1, keepdims=True)
                jl = jnp.min(jnp.where(fv == mn, lane512, 512), axis=1,
                             keepdims=True)
                gsel = jnp.sum(jnp.where(lane512 == jl, fi_ref[...], 0),
                               axis=1, keepdims=True)
                cm = jnp.max(bd, axis=1, keepdims=True)
                acc = mn < cm
                pk = jnp.min(jnp.where(bd == cm, laneK, K), axis=1,
                             keepdims=True)
                hit = (laneK == pk) & acc
                half = (jl >= 256).astype(jnp.int32)
                gidx = 2 * p * S + half * S + gsel * 256 + (jl & 255)
                bd = jnp.where(hit, mn, bd)
                bi = jnp.where(hit, gidx, bi)
                m_ref[...] = jnp.where(lane512 == jl, INF, fv)
            return bd, bi

        bd, bi = lax.fori_loop(0, (mx2 + 1) // 2, step2, (bd, bi))

        def step3(i, st):
            bd, bi = st
            for _ in range(2):
                dm = dm_ref[...]
                mn = jnp.min(dm, axis=1, keepdims=True)
                jl = jnp.min(jnp.where(dm == mn, laneS2, S2), axis=1,
                             keepdims=True)
                cm = jnp.max(bd, axis=1, keepdims=True)
                acc = mn < cm
                pk = jnp.min(jnp.where(bd == cm, laneK, K), axis=1,
                             keepdims=True)
                hit = (laneK == pk) & acc
                bd = jnp.where(hit, mn, bd)
                bi = jnp.where(hit, 2 * p * S + jl, bi)
                dm_ref[...] = jnp.where(laneS2 == jl, INF, dm)
            return bd, bi

        def cleanup(args):
            bd, bi = args
            fi0 = fi_ref[...]
            parts = [fi0[:, 0:256]] * NG + [fi0[:, 256:512]] * NG
            fi_t = jnp.concatenate(parts, axis=1)
            giota = (laneS2 & (S - 1)) >> 8
            dm_ref[...] = jnp.where(giota != fi_t, dm_ref[...], INF)
            return lax.fori_loop(0, (mxm + 1) // 2, step3, (bd, bi))

        return lax.cond(mxm > 0, cleanup, lambda args: args, (bd, bi))

    bd0 = jnp.full((BQ, K), INF, jnp.float32)
    bi0 = jnp.zeros((BQ, K), jnp.int32)
    bd, bi = lax.fori_loop(0, NP, pass_b, (bd0, bi0))

    d = jnp.sqrt(jnp.maximum(bd + x2, 1e-12))
    temp = t_ref[0, 0]
    logits = -d / temp
    mx = jnp.max(logits, axis=1, keepdims=True)
    e = jnp.exp(logits - mx)
    w = e / jnp.sum(e, axis=1, keepdims=True)
    w_ref[...] = w
    idx_ref[...] = bi


def _tc_select(x, st3, temp11):
    return pl.pallas_call(
        _tc_select_body,
        grid=(B // BQ,),
        in_specs=[
            pl.BlockSpec((BQ, D), lambda i: (i, 0)),
            pl.BlockSpec((NCH, D, S), lambda i: (0, 0, 0)),
            pl.BlockSpec(memory_space=pltpu.SMEM),
        ],
        out_specs=[
            pl.BlockSpec((BQ, K), lambda i: (i, 0)),
            pl.BlockSpec((BQ, K), lambda i: (i, 0)),
        ],
        out_shape=[
            jax.ShapeDtypeStruct((B, K), jnp.float32),
            jax.ShapeDtypeStruct((B, K), jnp.int32),
        ],
        scratch_shapes=[
            pltpu.VMEM((BQ, 2 * S), jnp.float32),
            pltpu.VMEM((BQ, 512), jnp.float32),
            pltpu.VMEM((BQ, 512), jnp.int32),
            pltpu.VMEM((NCH, 1, S), jnp.float32),
        ],
        compiler_params=pltpu.CompilerParams(
            vmem_limit_bytes=100 * 1024 * 1024,
        ),
    )(x, st3, temp11)


def _sc_combine(w_flat, idx_flat, labels):
    info = plsc.get_sparse_core_info()
    nc, ns = info.num_cores, info.num_subcores
    nw = nc * ns
    rw = B // nw
    rb = 32
    nblk = rw // rb
    mesh = plsc.VectorSubcoreMesh(core_axis_name="c", subcore_axis_name="s")
    cp = pltpu.CompilerParams()
    if "needs_layout_passes" in pltpu.CompilerParams.__dataclass_fields__:
        cp = dataclasses.replace(cp, needs_layout_passes=False)

    @functools.partial(
        pl.kernel,
        mesh=mesh,
        compiler_params=cp,
        out_type=jax.ShapeDtypeStruct((B * NCLS,), jnp.float32),
        scratch_types=[
            pltpu.VMEM((N,), jnp.int32),
            pltpu.VMEM((rb * K,), jnp.int32),
            pltpu.VMEM((rb * K,), jnp.float32),
            pltpu.VMEM((rb * NCLS,), jnp.float32),
        ],
    )
    def sc_kernel(w_hbm, idx_hbm, lab_hbm, out_hbm, labv, idxv, wv, pv):
        wid = lax.axis_index("s") * nc + lax.axis_index("c")
        pltpu.sync_copy(lab_hbm, labv)
        zero16 = jnp.zeros((16,), jnp.float32)

        def do_block(blk, _):
            base = (wid * rw + blk * rb)

            pltpu.sync_copy(idx_hbm.at[pl.ds(base * K, rb * K)], idxv)
            pltpu.sync_copy(w_hbm.at[pl.ds(base * K, rb * K)], wv)

            def zero_row(i, _):
                pv[pl.ds(i * 16, 16)] = zero16
                return 0

            lax.fori_loop(0, (rb * NCLS) // 16, zero_row, 0)

            def do_row(r, _):
                for g in range(K // 16):
                    iv = idxv[pl.ds(r * K + g * 16, 16)]
                    lab = plsc.load_gather(labv, [iv])
                    wvec = wv[pl.ds(r * K + g * 16, 16)]
                    flat = lab + r * NCLS
                    plsc.addupdate_scatter(pv, [flat], wvec)
                return 0

            lax.fori_loop(0, rb, do_row, 0)
            pltpu.sync_copy(pv, out_hbm.at[pl.ds(base * NCLS, rb * NCLS)])
            return 0

        lax.fori_loop(0, nblk, do_block, 0)

    return sc_kernel(w_flat, idx_flat, labels)


def kernel(x, support_embeddings, support_labels, temperature):
    sp = jnp.pad(support_embeddings, ((0, NPAD - N), (0, 0)))
    st3 = sp.reshape(NCH, S, D).transpose(0, 2, 1)
    temp11 = jnp.reshape(temperature, (1, 1)).astype(jnp.float32)
    w, idx = _tc_select(x, st3, temp11)
    probs_flat = _sc_combine(w.reshape(-1), idx.reshape(-1), support_labels)
    return probs_flat.reshape(B, NCLS)
